# baseline (speedup 1.0000x reference)
import functools

import numpy as np
import jax
import jax.numpy as jnp
from jax.experimental import pallas as pl
from jax.experimental.pallas import tpu as pltpu

_SCALES = (1, 2, 4, 8, 16, 32)
_D = 64
_K = 8192
_KC = 1024
_HP = jax.lax.Precision.HIGHEST


def _cubic_resize_mat(n_in, n_out):
    scale = n_out / n_in
    out = np.zeros((n_out, n_in), np.float64)
    for i in range(n_out):
        x = (i + 0.5) / scale - 0.5
        for j in range(n_in):
            t = abs(x - j)
            if t <= 1.0:
                w = 1.5 * t**3 - 2.5 * t**2 + 1.0
            elif t < 2.0:
                w = -0.5 * t**3 + 2.5 * t**2 - 4.0 * t + 2.0
            else:
                w = 0.0
            out[i, j] = w
        s = out[i].sum()
        if s != 0.0:
            out[i] /= s
    return out


def _area_mat(s, full=32):
    out = np.zeros((s, full), np.float64)
    r = full // s
    for i in range(s):
        out[i, i * r:(i + 1) * r] = 1.0 / r
    return out


@functools.lru_cache(maxsize=1)
def _consts():
    ups = []
    for k in range(1, 6):
        r = _cubic_resize_mat(_SCALES[k - 1], _SCALES[k])
        ups.append(np.kron(r, r).astype(np.float32))
    parts = []
    for s in _SCALES[:-1]:
        r = _cubic_resize_mat(s, 32)
        parts.append(np.kron(r, r).astype(np.float32))
    downs = []
    for s in _SCALES[:-1]:
        a = _area_mat(s)
        downs.append(np.kron(a, a).astype(np.float32))
    masks = np.zeros((6, 9, 1024), np.float32)
    for k, s in enumerate(_SCALES):
        for t in range(9):
            dy, dx = t // 3 - 1, t % 3 - 1
            for p in range(s * s):
                i, j = p // s, p % s
                if 0 <= i + dy < s and 0 <= j + dx < s:
                    masks[k, t, p] = 1.0
    return ups, parts, downs, masks


def _shift_cols(x, o):
    if o == 0:
        return x
    z = jnp.zeros((x.shape[0], abs(o)), x.dtype)
    if o > 0:
        return jnp.concatenate([x[:, o:], z], axis=1)
    return jnp.concatenate([z, x[:, :o]], axis=1)


def _conv3x3(x, w9, bias, masks_k, s):
    n = s * s
    y = jnp.broadcast_to(bias, (_D, n))
    for t in range(9):
        dy, dx = t // 3 - 1, t % 3 - 1
        if s == 1 and t != 4:
            continue
        xs = _shift_cols(x, dy * s + dx)
        if dx != 0:
            xs = xs * masks_k[t, 0:n][None, :]
        y = y + jax.lax.dot_general(
            w9[t].astype(jnp.bfloat16), xs.astype(jnp.bfloat16),
            (((1,), (0,)), ((), ())), preferred_element_type=jnp.float32)
    return y


def _body(z_ref, cb_ref, phi_ref, phib_ref, pq_ref, pqb_ref,
          u1, u2, u3, u4, u5, p1, p2, p3, p4, p5, d1, d2, d3, d4, d5,
          mask_ref, part_out, idx_out, zf_out, commit_out):
    f32 = jnp.float32
    z = z_ref[:]
    cb = cb_ref[:]
    cb_sq = jnp.sum(cb * cb, axis=1)
    ups = [None, u1, u2, u3, u4, u5]
    parts = [p1, p2, p3, p4, p5]
    downs = [d1, d2, d3, d4, d5]
    mask32 = mask_ref[5]
    commit = jnp.zeros((), f32)
    f_hat = None
    for k, s in enumerate(_SCALES):
        n = s * s
        if k == 0:
            f_up = jnp.zeros((_D, 1), f32)
        else:
            f_up = jax.lax.dot_general(f_hat, ups[k][:],
                                       (((1,), (1,)), ((), ())), precision=_HP)
        if s == 32:
            down = z
        else:
            down = jax.lax.dot_general(z, downs[k][:],
                                       (((1,), (1,)), ((), ())), precision=_HP)
        res = down - f_up
        res_t = res.T
        run_min = jnp.full((n, 1), jnp.inf, f32)
        run_idx = jnp.zeros((n, 1), jnp.int32)
        res_bf = res_t.astype(jnp.bfloat16)
        for c in range(_K // _KC):
            cb_c = cb[c * _KC:(c + 1) * _KC]
            prod = jax.lax.dot_general(
                res_bf, cb_c.astype(jnp.bfloat16),
                (((1,), (1,)), ((), ())),
                preferred_element_type=jnp.float32)
            dists = (-2.0) * prod
            dists = dists + cb_sq[c * _KC:(c + 1) * _KC][None, :]
            m = jnp.min(dists, axis=1, keepdims=True)
            iota = jax.lax.broadcasted_iota(jnp.int32, (n, _KC), 1)
            a = jnp.min(jnp.where(dists == m, iota, _KC), axis=1,
                        keepdims=True) + c * _KC
            better = m < run_min
            run_min = jnp.where(better, m, run_min)
            run_idx = jnp.where(better, a, run_idx)
        zq_t = jnp.zeros((n, _D), f32)
        for c in range(_K // _KC):
            iota = jax.lax.broadcasted_iota(jnp.int32, (n, _KC), 1) + c * _KC
            oh = (iota == run_idx).astype(f32)
            zq_t = zq_t + jnp.dot(oh, cb[c * _KC:(c + 1) * _KC], precision=_HP)
        commit = commit + jnp.mean((res_t - zq_t) ** 2)
        zq_in = res_t + (zq_t - res_t)
        y = _conv3x3(zq_in.T, phi_ref[k], phib_ref[k], mask_ref[k], s)
        f_hat = f_up + y
        if s == 32:
            pu = f_hat
        else:
            pu = jax.lax.dot_general(f_hat, parts[k][:],
                                     (((1,), (1,)), ((), ())), precision=_HP)
        partial = _conv3x3(pu, pq_ref[:], pqb_ref[:], mask32, 32)
        part_out[k] = partial
        idx_out[k, 0:n] = run_idx[:, 0]
        zf_out[k, 0:n, :] = res_t
    commit_out[:, :] = commit[None, None]


def kernel(z_e, codebook, phi_w, phi_b, pq_w, pq_b):
    f32 = jnp.float32
    z_flat = z_e.reshape(_D, 1024)
    phi_mat = jnp.transpose(phi_w, (0, 3, 4, 1, 2)).reshape(6, 9, _D, _D)
    phi_bias = phi_b.reshape(6, _D, 1)
    pq_mat = jnp.transpose(pq_w, (2, 3, 0, 1)).reshape(9, _D, _D)
    pq_bias = pq_b.reshape(_D, 1)
    ups, parts, downs, masks = _consts()
    out_shape = (
        jax.ShapeDtypeStruct((6, _D, 1024), f32),
        jax.ShapeDtypeStruct((6, 1024), jnp.int32),
        jax.ShapeDtypeStruct((6, 1024, _D), f32),
        jax.ShapeDtypeStruct((1, 1), f32),
    )
    part_buf, idx_buf, zf_buf, commit = pl.pallas_call(
        _body,
        out_shape=out_shape,
        compiler_params=pltpu.CompilerParams(
            vmem_limit_bytes=100 * 1024 * 1024),
    )(z_flat, codebook, phi_mat, phi_bias, pq_mat, pq_bias,
      *ups, *parts, *downs, masks)
    partials = tuple(part_buf[k].reshape(_D, 32, 32) for k in range(6))
    indices = tuple(idx_buf[k, :s * s] for k, s in enumerate(_SCALES))
    zfs = tuple(zf_buf[k, :s * s, :] for k, s in enumerate(_SCALES))
    return (partials[-1], indices, partials, commit[0, 0], zfs)


# speedup vs baseline: 1.5490x; 1.5490x over previous
import functools

import numpy as np
import jax
import jax.numpy as jnp
from jax.experimental import pallas as pl
from jax.experimental.pallas import tpu as pltpu

_SCALES = (1, 2, 4, 8, 16, 32)
_D = 64
_K = 8192
_KC = 1024
_HP = jax.lax.Precision.HIGHEST


def _cubic_resize_mat(n_in, n_out):
    scale = n_out / n_in
    out = np.zeros((n_out, n_in), np.float64)
    for i in range(n_out):
        x = (i + 0.5) / scale - 0.5
        for j in range(n_in):
            t = abs(x - j)
            if t <= 1.0:
                w = 1.5 * t**3 - 2.5 * t**2 + 1.0
            elif t < 2.0:
                w = -0.5 * t**3 + 2.5 * t**2 - 4.0 * t + 2.0
            else:
                w = 0.0
            out[i, j] = w
        s = out[i].sum()
        if s != 0.0:
            out[i] /= s
    return out


def _area_mat(s, full=32):
    out = np.zeros((s, full), np.float64)
    r = full // s
    for i in range(s):
        out[i, i * r:(i + 1) * r] = 1.0 / r
    return out


@functools.lru_cache(maxsize=1)
def _consts():
    ups = []
    for k in range(1, 6):
        r = _cubic_resize_mat(_SCALES[k - 1], _SCALES[k])
        ups.append(np.kron(r, r).astype(np.float32))
    parts = []
    for s in _SCALES[:-1]:
        r = _cubic_resize_mat(s, 32)
        parts.append(np.kron(r, r).astype(np.float32))
    downs = []
    for s in _SCALES[:-1]:
        a = _area_mat(s)
        downs.append(np.kron(a, a).astype(np.float32))
    masks = np.zeros((6, 9, 1024), np.float32)
    for k, s in enumerate(_SCALES):
        for t in range(9):
            dy, dx = t // 3 - 1, t % 3 - 1
            for p in range(s * s):
                i, j = p // s, p % s
                if 0 <= i + dy < s and 0 <= j + dx < s:
                    masks[k, t, p] = 1.0
    return ups, parts, downs, masks


def _shift_cols(x, o):
    if o == 0:
        return x
    z = jnp.zeros((x.shape[0], abs(o)), x.dtype)
    if o > 0:
        return jnp.concatenate([x[:, o:], z], axis=1)
    return jnp.concatenate([z, x[:, :o]], axis=1)


def _conv3x3(x, w9, bias, masks_k, s):
    n = s * s
    y = jnp.broadcast_to(bias, (_D, n))
    for t in range(9):
        dy, dx = t // 3 - 1, t % 3 - 1
        if s == 1 and t != 4:
            continue
        xs = _shift_cols(x, dy * s + dx)
        if dx != 0:
            xs = xs * masks_k[t, 0:n][None, :]
        y = y + jax.lax.dot_general(
            w9[t].astype(jnp.bfloat16), xs.astype(jnp.bfloat16),
            (((1,), (0,)), ((), ())), preferred_element_type=jnp.float32)
    return y


def _body(z_ref, cb_ref, phi_ref, phib_ref, pq_ref, pqb_ref,
          u1, u2, u3, u4, u5, p1, p2, p3, p4, p5, d1, d2, d3, d4, d5,
          mask_ref, part_out, idx_out, zf_out, commit_out):
    f32 = jnp.float32
    z = z_ref[:]
    cb = cb_ref[:]
    cb_sq = jnp.sum(cb * cb, axis=1)
    ups = [None, u1, u2, u3, u4, u5]
    parts = [p1, p2, p3, p4, p5]
    downs = [d1, d2, d3, d4, d5]
    mask32 = mask_ref[5]
    commit = jnp.zeros((), f32)
    f_hat = None
    for k, s in enumerate(_SCALES):
        n = s * s
        if k == 0:
            f_up = jnp.zeros((_D, 1), f32)
        else:
            f_up = jax.lax.dot_general(f_hat, ups[k][:],
                                       (((1,), (1,)), ((), ())), precision=_HP)
        if s == 32:
            down = z
        else:
            down = jax.lax.dot_general(z, downs[k][:],
                                       (((1,), (1,)), ((), ())), precision=_HP)
        res = down - f_up
        res_t = res.T
        run_min = jnp.full((n, 1), jnp.inf, f32)
        run_idx = jnp.zeros((n, 1), jnp.int32)
        res_bf = res_t.astype(jnp.bfloat16)
        for c in range(_K // _KC):
            cb_c = cb[c * _KC:(c + 1) * _KC]
            prod = jax.lax.dot_general(
                res_bf, cb_c.astype(jnp.bfloat16),
                (((1,), (1,)), ((), ())),
                preferred_element_type=jnp.float32)
            dists = (-2.0) * prod
            dists = dists + cb_sq[c * _KC:(c + 1) * _KC][None, :]
            m = jnp.min(dists, axis=1, keepdims=True)
            iota = jax.lax.broadcasted_iota(jnp.int32, (n, _KC), 1)
            a = jnp.min(jnp.where(dists == m, iota, _KC), axis=1,
                        keepdims=True) + c * _KC
            better = m < run_min
            run_min = jnp.where(better, m, run_min)
            run_idx = jnp.where(better, a, run_idx)
        zq_t = jnp.zeros((n, _D), f32)
        for c in range(_K // _KC):
            iota = jax.lax.broadcasted_iota(jnp.int32, (n, _KC), 1) + c * _KC
            oh = (iota == run_idx).astype(jnp.bfloat16)
            zq_t = zq_t + jax.lax.dot_general(
                oh, cb[c * _KC:(c + 1) * _KC].astype(jnp.bfloat16),
                (((1,), (0,)), ((), ())), preferred_element_type=f32)
        commit = commit + jnp.mean((res_t - zq_t) ** 2)
        zq_in = res_t + (zq_t - res_t)
        y = _conv3x3(zq_in.T, phi_ref[k], phib_ref[k], mask_ref[k], s)
        f_hat = f_up + y
        if s == 32:
            pu = f_hat
        else:
            pu = jax.lax.dot_general(f_hat, parts[k][:],
                                     (((1,), (1,)), ((), ())), precision=_HP)
        partial = _conv3x3(pu, pq_ref[:], pqb_ref[:], mask32, 32)
        part_out[k] = partial
        idx_out[k, 0:n] = run_idx[:, 0]
        zf_out[k, 0:n, :] = res_t
    commit_out[:, :] = commit[None, None]


def kernel(z_e, codebook, phi_w, phi_b, pq_w, pq_b):
    f32 = jnp.float32
    z_flat = z_e.reshape(_D, 1024)
    phi_mat = jnp.transpose(phi_w, (0, 3, 4, 1, 2)).reshape(6, 9, _D, _D)
    phi_bias = phi_b.reshape(6, _D, 1)
    pq_mat = jnp.transpose(pq_w, (2, 3, 0, 1)).reshape(9, _D, _D)
    pq_bias = pq_b.reshape(_D, 1)
    ups, parts, downs, masks = _consts()
    out_shape = (
        jax.ShapeDtypeStruct((6, _D, 1024), f32),
        jax.ShapeDtypeStruct((6, 1024), jnp.int32),
        jax.ShapeDtypeStruct((6, 1024, _D), f32),
        jax.ShapeDtypeStruct((1, 1), f32),
    )
    part_buf, idx_buf, zf_buf, commit = pl.pallas_call(
        _body,
        out_shape=out_shape,
        compiler_params=pltpu.CompilerParams(
            vmem_limit_bytes=100 * 1024 * 1024),
    )(z_flat, codebook, phi_mat, phi_bias, pq_mat, pq_bias,
      *ups, *parts, *downs, masks)
    partials = tuple(part_buf[k].reshape(_D, 32, 32) for k in range(6))
    indices = tuple(idx_buf[k, :s * s] for k, s in enumerate(_SCALES))
    zfs = tuple(zf_buf[k, :s * s, :] for k, s in enumerate(_SCALES))
    return (partials[-1], indices, partials, commit[0, 0], zfs)
